# baseline (speedup 1.0000x reference)
import functools

import jax
import jax.numpy as jnp
from jax import lax
from jax.experimental import pallas as pl
from jax.experimental.pallas import tpu as pltpu
from jax.experimental.pallas import tpu_sc as plsc

VOCAB = 1000000
EMBED = 64
BATCH = 4096
NUM_NEG = 64

_NC = 2
_NS = 16
_NW = _NC * _NS
_BPW = BATCH // _NW


def _sc_gather(head_table, tail_table, bias_table, head_idx, tail_idx, neg_idx):
  mesh = plsc.VectorSubcoreMesh(core_axis_name="c", subcore_axis_name="s")

  @functools.partial(
      pl.kernel,
      mesh=mesh,
      compiler_params=pltpu.CompilerParams(use_tc_tiling_on_sc=False),
      out_type=[
          jax.ShapeDtypeStruct((BATCH, EMBED), jnp.float32),
          jax.ShapeDtypeStruct((BATCH, EMBED), jnp.float32),
          jax.ShapeDtypeStruct((BATCH, 1), jnp.float32),
          jax.ShapeDtypeStruct((NUM_NEG, EMBED), jnp.float32),
      ],
      scratch_types=[
          pltpu.VMEM((_BPW,), jnp.int32),
          pltpu.VMEM((_BPW,), jnp.int32),
          pltpu.VMEM((_BPW, EMBED), jnp.float32),
          pltpu.VMEM((_BPW, EMBED), jnp.float32),
          pltpu.VMEM((_BPW, 1), jnp.float32),
          pltpu.VMEM((NUM_NEG,), jnp.int32),
          pltpu.VMEM((NUM_NEG, EMBED), jnp.float32),
          pltpu.SemaphoreType.DMA,
          pltpu.SemaphoreType.DMA,
          pltpu.SemaphoreType.DMA,
          pltpu.SemaphoreType.DMA,
      ],
  )
  def k(head_hbm, tail_hbm, bias_hbm, hidx_hbm, tidx_hbm, nidx_hbm,
        head_out, tail_out, bias_out, neg_out,
        hidx_v, tidx_v, hrows_v, trows_v, brows_v, nidx_v, nrows_v,
        sem_h, sem_t, sem_b, sem_n):
    wid = lax.axis_index("s") * _NC + lax.axis_index("c")
    base = wid * _BPW
    pltpu.sync_copy(hidx_hbm.at[pl.ds(base, _BPW)], hidx_v)
    pltpu.sync_copy(tidx_hbm.at[pl.ds(base, _BPW)], tidx_v)
    ch = pltpu.async_copy(head_hbm.at[hidx_v], hrows_v, sem_h)
    ct = pltpu.async_copy(tail_hbm.at[tidx_v], trows_v, sem_t)
    cb = pltpu.async_copy(bias_hbm.at[tidx_v], brows_v, sem_b)

    @pl.when(wid == 0)
    def _():
      pltpu.sync_copy(nidx_hbm, nidx_v)
      pltpu.async_copy(tail_hbm.at[nidx_v], nrows_v, sem_n).wait()
      pltpu.sync_copy(nrows_v, neg_out)

    ch.wait()
    pltpu.sync_copy(hrows_v, head_out.at[pl.ds(base, _BPW)])
    ct.wait()
    pltpu.sync_copy(trows_v, tail_out.at[pl.ds(base, _BPW)])
    cb.wait()
    pltpu.sync_copy(brows_v, bias_out.at[pl.ds(base, _BPW)])

  return k(head_table, tail_table, bias_table, head_idx, tail_idx, neg_idx)


_LN2 = 0.6931471805599453


def _tc_body(head_ref, tail_ref, bias_ref, neg_ref, rel_ref, out_ref):
  ex = head_ref[...] + rel_ref[...]
  pos = jnp.sum(tail_ref[...] * ex, axis=1, keepdims=True) + bias_ref[...]
  pos_loss_c = jnp.log(0.5 * (1.0 + jnp.exp(-pos)))
  neg = lax.dot_general(ex, neg_ref[...],
                        dimension_numbers=(((1,), (1,)), ((), ())),
                        preferred_element_type=jnp.float32)
  neg = neg + bias_ref[...]
  neg_loss_c = jnp.sum(jnp.log(0.5 * (1.0 + jnp.exp(neg))), axis=1, keepdims=True)
  out_ref[0, 0] = (jnp.sum(pos_loss_c + neg_loss_c) * (1.0 / BATCH)
                   + (NUM_NEG + 1) * _LN2)


def _tc_loss(head_vec, tail_vec, bias, neg_vec, relation_vec):
  return pl.pallas_call(
      _tc_body,
      out_shape=jax.ShapeDtypeStruct((1, 1), jnp.float32),
      in_specs=[
          pl.BlockSpec(memory_space=pltpu.VMEM),
          pl.BlockSpec(memory_space=pltpu.VMEM),
          pl.BlockSpec(memory_space=pltpu.VMEM),
          pl.BlockSpec(memory_space=pltpu.VMEM),
          pl.BlockSpec(memory_space=pltpu.VMEM),
      ],
      out_specs=pl.BlockSpec(memory_space=pltpu.SMEM),
  )(head_vec, tail_vec, bias, neg_vec, relation_vec)


def kernel(head_table, tail_table, relation_vec, bias_table, batch_idxs, neg_idx):
  head_idx = batch_idxs[:, 0]
  tail_idx = batch_idxs[:, 1]
  head_vec, tail_vec, bias, neg_vec = _sc_gather(
      head_table, tail_table, bias_table, head_idx, tail_idx, neg_idx)
  loss = _tc_loss(head_vec, tail_vec, bias, neg_vec, relation_vec)
  return loss[0, 0]
